# Initial kernel scaffold; baseline (speedup 1.0000x reference)
import jax
import jax.numpy as jnp
from jax.experimental import pallas as pl


def kernel(x, W1, b1, W2, b2, log_temperature, mu, sigma, Wl, bl, Wp, bp):
    raise NotImplementedError("write your pallas kernel here")


# speedup vs baseline: 37.3342x; 37.3342x over previous
import functools

import jax
import jax.numpy as jnp
from jax.experimental import pallas as pl

N = 2048
INPUT_DIM = 16
HIDDEN_DIM = 64
LIFT_K = 16
D_MODEL = 256
SEL_K = 8.0
LAM = 0.5
K_TOP = 16
ANCHOR_DIM = INPUT_DIM + 2
ROW_BLK = 512

_DNT = (((1,), (1,)), ((), ()))


def _encoder_kernel(x_ref, w1_ref, b1_ref, w2p_ref, scal_ref, mu_ref,
                    sigma_ref, wl_ref, bl_ref, wp_ref, bp_ref,
                    tokens_ref, y_ref):
    x = x_ref[0]

    hidden = jnp.maximum(
        jnp.dot(x, w1_ref[...], preferred_element_type=jnp.float32)
        + b1_ref[0][None, :], 0.0)
    b2 = scal_ref[0, 1]
    saliency = (jnp.dot(hidden, w2p_ref[...],
                        preferred_element_type=jnp.float32) + b2)[:, 0]

    sq = jnp.sum(x * x, axis=1)
    mins = []
    for i in range(N // ROW_BLK):
        xb = x[i * ROW_BLK:(i + 1) * ROW_BLK]
        g = jax.lax.dot_general(xb, x, _DNT, preferred_element_type=jnp.float32)
        d = sq[i * ROW_BLK:(i + 1) * ROW_BLK][:, None] + sq[None, :] - 2.0 * g
        d = jnp.maximum(d, 0.0)
        rows = jax.lax.broadcasted_iota(jnp.int32, (ROW_BLK, N), 0) + i * ROW_BLK
        cols = jax.lax.broadcasted_iota(jnp.int32, (ROW_BLK, N), 1)
        d = jnp.where(rows == cols, d + 1e9, d)
        mins.append(jnp.min(d, axis=1))
    dmin = jnp.concatenate(mins)

    knn = jnp.sqrt(jnp.maximum(dmin, 0.0))
    density = 1.0 / (1.0 + knn)

    f = jnp.concatenate(
        [x, knn[:, None], density[:, None], saliency[:, None]], axis=1)
    fnorm = jnp.sqrt(jnp.sum(f * f, axis=1))
    fn = f / (fnorm + 1e-8)[:, None]

    temp = scal_ref[0, 0]
    y = jax.nn.sigmoid((saliency / (2.0 * LAM) - 0.5) / temp)
    budget = jnp.maximum(jnp.sum(y), 1e-6)
    y = y * jnp.minimum(SEL_K / budget, 1.0)
    ypad = jnp.where(
        jax.lax.broadcasted_iota(jnp.int32, (128, N), 0) == 0,
        y[None, :], 0.0)
    ovs = []
    for i in range(N // ROW_BLK):
        fb = fn[i * ROW_BLK:(i + 1) * ROW_BLK]
        sim = jax.lax.dot_general(fb, fn, _DNT,
                                  preferred_element_type=jnp.float32)
        ov = jax.lax.dot_general(sim, ypad, _DNT,
                                 preferred_element_type=jnp.float32)
        ovs.append(ov[:, 0])
    overlap = jnp.concatenate(ovs)
    y = y / (1.0 + overlap)
    budget = jnp.maximum(jnp.sum(y), 1e-6)
    y = y * jnp.minimum(SEL_K / budget, 1.0)
    y_ref[0, 0] = y

    iota = jax.lax.broadcasted_iota(jnp.int32, (N,), 0)
    y_work = y
    onehot_rows = []
    for _ in range(K_TOP):
        m = jnp.max(y_work)
        idx = jnp.min(jnp.where(y_work == m, iota, N))
        hit = iota == idx
        onehot_rows.append(jnp.where(hit, 1.0, 0.0)[None, :])
        y_work = jnp.where(hit, -1.0, y_work)
    onehot = jnp.concatenate(onehot_rows, axis=0)

    zf = jnp.concatenate([x, knn[:, None], density[:, None]], axis=1)
    z = (zf - mu_ref[0][None, :]) / sigma_ref[0][None, :]
    z_sel = jnp.dot(onehot, z, preferred_element_type=jnp.float32)
    lifted = jnp.tanh(
        jnp.dot(z_sel, wl_ref[...], preferred_element_type=jnp.float32)
        + bl_ref[0][None, :])
    tokens_ref[0] = (
        jnp.dot(lifted, wp_ref[...], preferred_element_type=jnp.float32)
        + bp_ref[0][None, :])


@functools.partial(jax.jit, static_argnames=("interpret",))
def kernel(x, W1, b1, W2, b2, log_temperature, mu, sigma, Wl, bl, Wp, bp,
           interpret=False):
    B = x.shape[0]
    temp = jnp.clip(jnp.exp(log_temperature), 0.1, 10.0)
    scal = jnp.stack([temp, b2[0]]).reshape(1, 2).astype(jnp.float32)
    w2p = jnp.pad(W2, ((0, 0), (0, 127)))

    full = lambda *shape: pl.BlockSpec(shape, lambda b: (0,) * len(shape))
    grid_spec = pl.GridSpec(
        grid=(B,),
        in_specs=[
            pl.BlockSpec((1, N, INPUT_DIM), lambda b: (b, 0, 0)),
            full(INPUT_DIM, HIDDEN_DIM),
            full(1, HIDDEN_DIM),
            full(HIDDEN_DIM, 128),
            full(1, 2),
            full(1, ANCHOR_DIM),
            full(1, ANCHOR_DIM),
            full(ANCHOR_DIM, LIFT_K),
            full(1, LIFT_K),
            full(LIFT_K, D_MODEL),
            full(1, D_MODEL),
        ],
        out_specs=[
            pl.BlockSpec((1, K_TOP, D_MODEL), lambda b: (b, 0, 0)),
            pl.BlockSpec((1, 1, N), lambda b: (b, 0, 0)),
        ],
    )
    tokens, y_star = pl.pallas_call(
        _encoder_kernel,
        grid_spec=grid_spec,
        out_shape=[
            jax.ShapeDtypeStruct((B, K_TOP, D_MODEL), jnp.float32),
            jax.ShapeDtypeStruct((B, 1, N), jnp.float32),
        ],
        interpret=interpret,
    )(x, W1, b1.reshape(1, HIDDEN_DIM), w2p, scal, mu.reshape(1, ANCHOR_DIM),
      sigma.reshape(1, ANCHOR_DIM), Wl, bl.reshape(1, LIFT_K), Wp,
      bp.reshape(1, D_MODEL))
    return (tokens, y_star.reshape(B, N))


# speedup vs baseline: 39.1929x; 1.0498x over previous
import functools

import jax
import jax.numpy as jnp
from jax.experimental import pallas as pl

N = 2048
INPUT_DIM = 16
HIDDEN_DIM = 64
LIFT_K = 16
D_MODEL = 256
SEL_K = 8.0
LAM = 0.5
K_TOP = 16
ANCHOR_DIM = INPUT_DIM + 2
ROW_BLK = 512

_DNT = (((1,), (1,)), ((), ()))


def _encoder_kernel(x_ref, w1_ref, b1_ref, w2p_ref, scal_ref, mu_ref,
                    sigma_ref, wl_ref, bl_ref, wp_ref, bp_ref,
                    tokens_ref, y_ref):
    x = x_ref[0]

    hidden = jnp.maximum(
        jnp.dot(x, w1_ref[...], preferred_element_type=jnp.float32)
        + b1_ref[0][None, :], 0.0)
    b2 = scal_ref[0, 1]
    saliency = (jnp.dot(hidden, w2p_ref[...],
                        preferred_element_type=jnp.float32) + b2)[:, 0]

    sq = jnp.sum(x * x, axis=1)
    colmrow = (jax.lax.broadcasted_iota(jnp.int32, (ROW_BLK, N), 1)
               - jax.lax.broadcasted_iota(jnp.int32, (ROW_BLK, N), 0))
    n_tiles = N // 128
    diag_tiles = ROW_BLK // 128
    mins = []
    for i in range(N // ROW_BLK):
        xb = x[i * ROW_BLK:(i + 1) * ROW_BLK]
        g = jax.lax.dot_general(xb, x, _DNT, preferred_element_type=jnp.float32)
        d = sq[i * ROW_BLK:(i + 1) * ROW_BLK][:, None] + sq[None, :] - 2.0 * g
        m = None
        for t in range(n_tiles):
            dt = d[:, t * 128:(t + 1) * 128]
            if diag_tiles * i <= t < diag_tiles * (i + 1):
                cmr = colmrow[:, t * 128:(t + 1) * 128]
                dt = jnp.where(cmr == i * ROW_BLK, 1e9, dt)
            m = dt if m is None else jnp.minimum(m, dt)
        mins.append(jnp.min(m, axis=1))
    dmin = jnp.concatenate(mins)

    knn = jnp.sqrt(jnp.maximum(dmin, 0.0))
    density = 1.0 / (1.0 + knn)

    f = jnp.concatenate(
        [x, knn[:, None], density[:, None], saliency[:, None]], axis=1)
    fnorm = jnp.sqrt(jnp.sum(f * f, axis=1))
    fn = f / (fnorm + 1e-8)[:, None]

    temp = scal_ref[0, 0]
    y = jax.nn.sigmoid((saliency / (2.0 * LAM) - 0.5) / temp)
    budget = jnp.maximum(jnp.sum(y), 1e-6)
    y = y * jnp.minimum(SEL_K / budget, 1.0)
    ypad = jnp.where(
        jax.lax.broadcasted_iota(jnp.int32, (8, N), 0) == 0,
        y[None, :], 0.0)
    ovs = []
    for i in range(N // ROW_BLK):
        fb = fn[i * ROW_BLK:(i + 1) * ROW_BLK]
        sim = jax.lax.dot_general(fb, fn, _DNT,
                                  preferred_element_type=jnp.float32)
        ov = jax.lax.dot_general(ypad, sim, _DNT,
                                 preferred_element_type=jnp.float32)
        ovs.append(ov[0, :])
    overlap = jnp.concatenate(ovs)
    y = y / (1.0 + overlap)
    budget = jnp.maximum(jnp.sum(y), 1e-6)
    y = y * jnp.minimum(SEL_K / budget, 1.0)
    y_ref[0, 0] = y

    iota = jax.lax.broadcasted_iota(jnp.int32, (N,), 0)
    y_work = y
    onehot_rows = []
    for _ in range(K_TOP):
        m = jnp.max(y_work)
        idx = jnp.min(jnp.where(y_work == m, iota, N))
        hit = iota == idx
        onehot_rows.append(jnp.where(hit, 1.0, 0.0)[None, :])
        y_work = jnp.where(hit, -1.0, y_work)
    onehot = jnp.concatenate(onehot_rows, axis=0)

    zf = jnp.concatenate([x, knn[:, None], density[:, None]], axis=1)
    z = (zf - mu_ref[0][None, :]) / sigma_ref[0][None, :]
    z_sel = jnp.dot(onehot, z, preferred_element_type=jnp.float32)
    lifted = jnp.tanh(
        jnp.dot(z_sel, wl_ref[...], preferred_element_type=jnp.float32)
        + bl_ref[0][None, :])
    tokens_ref[0] = (
        jnp.dot(lifted, wp_ref[...], preferred_element_type=jnp.float32)
        + bp_ref[0][None, :])


@functools.partial(jax.jit, static_argnames=("interpret",))
def kernel(x, W1, b1, W2, b2, log_temperature, mu, sigma, Wl, bl, Wp, bp,
           interpret=False):
    B = x.shape[0]
    temp = jnp.clip(jnp.exp(log_temperature), 0.1, 10.0)
    scal = jnp.stack([temp, b2[0]]).reshape(1, 2).astype(jnp.float32)
    w2p = jnp.pad(W2, ((0, 0), (0, 127)))

    full = lambda *shape: pl.BlockSpec(shape, lambda b: (0,) * len(shape))
    grid_spec = pl.GridSpec(
        grid=(B,),
        in_specs=[
            pl.BlockSpec((1, N, INPUT_DIM), lambda b: (b, 0, 0)),
            full(INPUT_DIM, HIDDEN_DIM),
            full(1, HIDDEN_DIM),
            full(HIDDEN_DIM, 128),
            full(1, 2),
            full(1, ANCHOR_DIM),
            full(1, ANCHOR_DIM),
            full(ANCHOR_DIM, LIFT_K),
            full(1, LIFT_K),
            full(LIFT_K, D_MODEL),
            full(1, D_MODEL),
        ],
        out_specs=[
            pl.BlockSpec((1, K_TOP, D_MODEL), lambda b: (b, 0, 0)),
            pl.BlockSpec((1, 1, N), lambda b: (b, 0, 0)),
        ],
    )
    tokens, y_star = pl.pallas_call(
        _encoder_kernel,
        grid_spec=grid_spec,
        out_shape=[
            jax.ShapeDtypeStruct((B, K_TOP, D_MODEL), jnp.float32),
            jax.ShapeDtypeStruct((B, 1, N), jnp.float32),
        ],
        interpret=interpret,
    )(x, W1, b1.reshape(1, HIDDEN_DIM), w2p, scal, mu.reshape(1, ANCHOR_DIM),
      sigma.reshape(1, ANCHOR_DIM), Wl, bl.reshape(1, LIFT_K), Wp,
      bp.reshape(1, D_MODEL))
    return (tokens, y_star.reshape(B, N))
